# baseline (speedup 1.0000x reference)
import jax
import jax.numpy as jnp
from jax import lax
from jax.experimental import pallas as pl
from jax.experimental.pallas import tpu as pltpu
from jax.experimental.pallas import tpu_sc as plsc

VOCAB = 100000
BATCH = 16384
D_BASIC = 300
D_MODIF = 100
D_OUT = D_BASIC + D_MODIF

G = 8
GB = D_BASIC // G + 1
GM = D_MODIF // G + 1

NUM_CORES = 2
NUM_SUBCORES = 16
NUM_WORKERS = NUM_CORES * NUM_SUBCORES
B_PER_W = BATCH // NUM_WORKERS
CHUNK = 128
N_CHUNKS = B_PER_W // CHUNK
LANES = 16
VPC = CHUNK // LANES


def _body(ids_hbm, basic_hbm, modif_hbm, out_hbm,
          ids_v, s_v, idxb, idxm, stage_b, stage_m, out_v, sem):
    wid = lax.axis_index("s") * NUM_CORES + lax.axis_index("c")
    base = wid * B_PER_W
    lane = lax.iota(jnp.int32, LANES)

    for ch in range(N_CHUNKS):
        off = base + ch * CHUNK
        pltpu.sync_copy(ids_hbm.at[pl.ds(off, CHUNK)], ids_v)

        for v in range(VPC):
            ids16 = ids_v[pl.ds(v * LANES, LANES)]
            s16 = (ids16 & 1) * 4
            s_v[pl.ds(v * LANES, LANES)] = s16
            gb16 = (ids16 * D_BASIC) >> 3
            gm16 = (ids16 * D_MODIF) >> 3
            for c in range(GB):
                idxb[c][pl.ds(v * LANES, LANES)] = gb16 + c
            for c in range(GM):
                idxm[c][pl.ds(v * LANES, LANES)] = gm16 + c

        copies = []
        for c in range(GB):
            copies.append(pltpu.async_copy(
                basic_hbm.at[idxb[c]], stage_b.at[pl.ds(c * CHUNK, CHUNK)],
                sem))
        for c in range(GM):
            copies.append(pltpu.async_copy(
                modif_hbm.at[idxm[c]], stage_m.at[pl.ds(c * CHUNK, CHUNK)],
                sem))
        for cp in copies:
            cp.wait()

        def row_body(jj, carry):
            s16 = plsc.load_gather(s_v, [jnp.full((LANES,), jj, jnp.int32)])
            j16 = jnp.full((LANES,), jj, jnp.int32)
            for grp in range(D_BASIC // LANES):
                sw = s16 + (lane + grp * LANES)
                val = plsc.load_gather(
                    stage_b, [((sw >> 3) * CHUNK) + j16, sw & 7])
                out_v[jj, pl.ds(grp * LANES, LANES)] = val
            swb = s16 + (lane + 288)
            vb = plsc.load_gather(
                stage_b, [((swb >> 3) * CHUNK) + j16, swb & 7])
            swm = jnp.maximum(s16 + (lane - 12), 0)
            vm = plsc.load_gather(
                stage_m, [((swm >> 3) * CHUNK) + j16, swm & 7])
            out_v[jj, pl.ds(288, LANES)] = jnp.where(lane < 12, vb, vm)
            for grp in range(6):
                sw = s16 + (lane + grp * LANES + 4)
                val = plsc.load_gather(
                    stage_m, [((sw >> 3) * CHUNK) + j16, sw & 7])
                out_v[jj, pl.ds(304 + grp * LANES, LANES)] = val
            return carry

        lax.fori_loop(0, CHUNK, row_body, 0)
        pltpu.sync_copy(out_v, out_hbm.at[pl.ds(off, CHUNK)])


@jax.jit
def kernel(word_ids, basic, modif_weight):
    ids = word_ids.astype(jnp.int32)
    basic_g = basic.reshape(VOCAB * D_BASIC // G, G)
    modif_g = modif_weight.reshape(VOCAB * D_MODIF // G, G)
    mesh = plsc.VectorSubcoreMesh(
        core_axis_name="c", subcore_axis_name="s",
        num_cores=NUM_CORES, num_subcores=NUM_SUBCORES)
    run = pl.kernel(
        _body,
        out_type=jax.ShapeDtypeStruct((BATCH, D_OUT), jnp.float32),
        mesh=mesh,
        scratch_types=[
            pltpu.VMEM((CHUNK,), jnp.int32),
            pltpu.VMEM((CHUNK,), jnp.int32),
            [pltpu.VMEM((CHUNK,), jnp.int32) for _ in range(GB)],
            [pltpu.VMEM((CHUNK,), jnp.int32) for _ in range(GM)],
            pltpu.VMEM((GB * CHUNK, G), jnp.float32),
            pltpu.VMEM((GM * CHUNK, G), jnp.float32),
            pltpu.VMEM((CHUNK, D_OUT), jnp.float32),
            pltpu.SemaphoreType.DMA,
        ],
        compiler_params=pltpu.CompilerParams(
            use_tc_tiling_on_sc=False, needs_layout_passes=False),
    )
    return run(ids, basic_g, modif_g)


# speedup vs baseline: 1.3481x; 1.3481x over previous
import jax
import jax.numpy as jnp
from jax import lax
from jax.experimental import pallas as pl
from jax.experimental.pallas import tpu as pltpu
from jax.experimental.pallas import tpu_sc as plsc

VOCAB = 100000
BATCH = 16384
D_BASIC = 300
D_MODIF = 100
D_OUT = D_BASIC + D_MODIF

G = 8
ROWS_B = VOCAB // G
ROWS_M = VOCAB // G

NUM_CORES = 2
NUM_SUBCORES = 16
NUM_WORKERS = NUM_CORES * NUM_SUBCORES
B_PER_W = BATCH // NUM_WORKERS
CHUNK = 128
N_CHUNKS = B_PER_W // CHUNK
LANES = 16
VPC = CHUNK // LANES
FB = 10


def _body(ids_hbm, basic_hbm, modif_hbm, out_hbm,
          ids_v, gbase, cidx, idxs, stage, out_v, sem):
    wid = lax.axis_index("s") * NUM_CORES + lax.axis_index("c")
    base = wid * B_PER_W
    lane = lax.iota(jnp.int32, LANES)

    for ch in range(N_CHUNKS):
        coff = base + ch * CHUNK
        pltpu.sync_copy(ids_hbm.at[pl.ds(coff, CHUNK)], ids_v)
        for v in range(VPC):
            ids16 = ids_v[pl.ds(v * LANES, LANES)]
            gbase[pl.ds(v * LANES, LANES)] = ids16 >> 3
            cidx[pl.ds(v * LANES, LANES)] = ids16 & 7

        def bank(it, table_hbm, f0_out, f0_tab):
            for u in range(FB):
                fbase = (it * FB + u + f0_tab) * ROWS_B
                for v in range(VPC):
                    idxs[u, pl.ds(v * LANES, LANES)] = (
                        gbase[pl.ds(v * LANES, LANES)]
                        + jnp.full((LANES,), fbase, jnp.int32))
            copies = [
                pltpu.async_copy(table_hbm.at[idxs.at[u]],
                                 stage.at[pl.ds(u * CHUNK, CHUNK)], sem)
                for u in range(FB)]
            for cp in copies:
                cp.wait()
            for u in range(FB):
                fo = it * FB + u + f0_out
                for v in range(VPC):
                    rows16 = lane + (u * CHUNK + v * LANES)
                    val = plsc.load_gather(
                        stage, [rows16, cidx[pl.ds(v * LANES, LANES)]])
                    out_v[fo, pl.ds(v * LANES, LANES)] = val

        def basic_body(it, carry):
            bank(it, basic_hbm, 0, 0)
            return carry

        def modif_body(it, carry):
            bank(it, modif_hbm, D_BASIC, 0)
            return carry

        lax.fori_loop(0, D_BASIC // FB, basic_body, 0)
        lax.fori_loop(0, D_MODIF // FB, modif_body, 0)
        pltpu.sync_copy(out_v, out_hbm.at[:, pl.ds(coff, CHUNK)])


@jax.jit
def kernel(word_ids, basic, modif_weight):
    ids = word_ids.astype(jnp.int32)
    basic_t = basic.T.reshape(D_BASIC * VOCAB // G, G)
    modif_t = modif_weight.T.reshape(D_MODIF * VOCAB // G, G)
    mesh = plsc.VectorSubcoreMesh(
        core_axis_name="c", subcore_axis_name="s",
        num_cores=NUM_CORES, num_subcores=NUM_SUBCORES)
    run = pl.kernel(
        _body,
        out_type=jax.ShapeDtypeStruct((D_OUT, BATCH), jnp.float32),
        mesh=mesh,
        scratch_types=[
            pltpu.VMEM((CHUNK,), jnp.int32),
            pltpu.VMEM((CHUNK,), jnp.int32),
            pltpu.VMEM((CHUNK,), jnp.int32),
            pltpu.VMEM((FB, CHUNK), jnp.int32),
            pltpu.VMEM((FB * CHUNK, G), jnp.float32),
            pltpu.VMEM((D_OUT, CHUNK), jnp.float32),
            pltpu.SemaphoreType.DMA,
        ],
        compiler_params=pltpu.CompilerParams(
            use_tc_tiling_on_sc=False, needs_layout_passes=False),
    )
    out_t = run(ids, basic_t, modif_t)
    return out_t.T


# speedup vs baseline: 1.6177x; 1.2000x over previous
import jax
import jax.numpy as jnp
from jax import lax
from jax.experimental import pallas as pl
from jax.experimental.pallas import tpu as pltpu
from jax.experimental.pallas import tpu_sc as plsc

VOCAB = 100000
BATCH = 16384
D_BASIC = 300
D_MODIF = 100
D_OUT = D_BASIC + D_MODIF

G = 8
ROWS_B = VOCAB // G
ROWS_M = VOCAB // G

NUM_CORES = 2
NUM_SUBCORES = 16
NUM_WORKERS = NUM_CORES * NUM_SUBCORES
B_PER_W = BATCH // NUM_WORKERS
CHUNK = 128
N_CHUNKS = B_PER_W // CHUNK
LANES = 16
VPC = CHUNK // LANES
FB = 10


def _body(ids_hbm, basic_hbm, modif_hbm, out_hbm,
          ids_v, gbase, cidx, idxs, stage, out_v, sems):
    wid = lax.axis_index("s") * NUM_CORES + lax.axis_index("c")
    base = wid * B_PER_W
    lane = lax.iota(jnp.int32, LANES)

    for ch in range(N_CHUNKS):
        coff = base + ch * CHUNK
        pltpu.sync_copy(ids_hbm.at[pl.ds(coff, CHUNK)], ids_v)
        for v in range(VPC):
            ids16 = ids_v[pl.ds(v * LANES, LANES)]
            gbase[pl.ds(v * LANES, LANES)] = ids16 >> 3
            cidx[pl.ds(v * LANES, LANES)] = ids16 & 7

        def fire(slot, table_hbm, fbank):
            for u in range(FB):
                fbase = (fbank + u) * ROWS_B
                for v in range(VPC):
                    idxs[slot][pl.ds(u * CHUNK + v * LANES, LANES)] = (
                        gbase[pl.ds(v * LANES, LANES)]
                        + jnp.full((LANES,), fbase, jnp.int32))
            return pltpu.async_copy(
                table_hbm.at[idxs[slot]], stage[slot], sems[slot])

        def assemble(slot, f0_out):
            for u in range(FB):
                fo = f0_out + u
                for v in range(VPC):
                    rows16 = lane + (u * CHUNK + v * LANES)
                    val = plsc.load_gather(
                        stage[slot],
                        [rows16, cidx[pl.ds(v * LANES, LANES)]])
                    out_v[fo, pl.ds(v * LANES, LANES)] = val

        def pair_body(table_hbm, f0_out):
            def body(it, carry):
                f = it * 2 * FB
                cp_a = fire(0, table_hbm, f)
                cp_b = fire(1, table_hbm, f + FB)
                cp_a.wait()
                assemble(0, f0_out + f)
                cp_b.wait()
                assemble(1, f0_out + f + FB)
                return carry
            return body

        lax.fori_loop(0, D_BASIC // (2 * FB), pair_body(basic_hbm, 0), 0)
        lax.fori_loop(0, D_MODIF // (2 * FB), pair_body(modif_hbm, D_BASIC), 0)
        pltpu.sync_copy(out_v, out_hbm.at[:, pl.ds(coff, CHUNK)])


@jax.jit
def kernel(word_ids, basic, modif_weight):
    ids = word_ids.astype(jnp.int32)
    basic_t = basic.T.reshape(D_BASIC * VOCAB // G, G)
    modif_t = modif_weight.T.reshape(D_MODIF * VOCAB // G, G)
    mesh = plsc.VectorSubcoreMesh(
        core_axis_name="c", subcore_axis_name="s",
        num_cores=NUM_CORES, num_subcores=NUM_SUBCORES)
    run = pl.kernel(
        _body,
        out_type=jax.ShapeDtypeStruct((D_OUT, BATCH), jnp.float32),
        mesh=mesh,
        scratch_types=[
            pltpu.VMEM((CHUNK,), jnp.int32),
            pltpu.VMEM((CHUNK,), jnp.int32),
            pltpu.VMEM((CHUNK,), jnp.int32),
            [pltpu.VMEM((FB * CHUNK,), jnp.int32) for _ in range(2)],
            [pltpu.VMEM((FB * CHUNK, G), jnp.float32) for _ in range(2)],
            pltpu.VMEM((D_OUT, CHUNK), jnp.float32),
            [pltpu.SemaphoreType.DMA for _ in range(2)],
        ],
        compiler_params=pltpu.CompilerParams(
            use_tc_tiling_on_sc=False, needs_layout_passes=False),
    )
    out_t = run(ids, basic_t, modif_t)
    return out_t.T
